# baseline (speedup 1.0000x reference)
import functools

import jax
import jax.numpy as jnp
from jax import lax
from jax.experimental import pallas as pl
from jax.experimental.pallas import tpu as pltpu
from jax.experimental.pallas import tpu_sc as plsc

_N = 10000
_E = 320000
_H = 128
_NW = 32
_EPT = _E // _NW
_CHUNK = 80
_NCH = _EPT // _CHUNK
_NG = _CHUNK // 16
_RPT = 624
_RTAIL = _N - 16 * _RPT
_DR = 80
_BN = 10000

_PREC = lax.Precision.HIGHEST


def _dot_t(x, w):
    return lax.dot_general(x, w, (((1,), (1,)), ((), ())), precision=_PREC,
                           preferred_element_type=jnp.float32)


def _pre_body(h_ref, w1a_ref, w1b_ref, b1_ref, a_ref, b_ref):
    hb = h_ref[...]
    a_ref[...] = _dot_t(hb, w1a_ref[...]) + b1_ref[...]
    b_ref[...] = _dot_t(hb, w1b_ref[...])


def _pre(h, w1, b1r):
    return pl.pallas_call(
        _pre_body,
        grid=(_N // _BN,),
        in_specs=[
            pl.BlockSpec((_BN, _H), lambda i: (i, 0)),
            pl.BlockSpec((_H, _H), lambda i: (0, 0)),
            pl.BlockSpec((_H, _H), lambda i: (0, 1)),
            pl.BlockSpec((1, _H), lambda i: (0, 0)),
        ],
        out_specs=[pl.BlockSpec((_BN, _H), lambda i: (i, 0))] * 2,
        out_shape=[jax.ShapeDtypeStruct((_N, _H), jnp.float32)] * 2,
    )(h, w1, w1, b1r)


def _post_body(h_ref, s_ref, d_ref, w2_ref, b2_ref, w3a_ref, w3b_ref,
               b3_ref, w4_ref, b4_ref, out_ref):
    s = s_ref[0] + s_ref[1]
    deg = d_ref[0] + d_ref[1]
    agg = _dot_t(s, w2_ref[...]) + deg * b2_ref[...]
    hb = h_ref[...]
    hupd = jnp.maximum(
        _dot_t(hb, w3a_ref[...]) + _dot_t(agg, w3b_ref[...]) + b3_ref[...],
        0.0)
    out_ref[...] = hb + _dot_t(hupd, w4_ref[...]) + b4_ref[...]


def _post(h, s, d, w2, b2r, w3, b3r, w4, b4r):
    wspec = pl.BlockSpec((_H, _H), lambda i: (0, 0))
    bspec = pl.BlockSpec((1, _H), lambda i: (0, 0))
    return pl.pallas_call(
        _post_body,
        grid=(_N // _BN,),
        in_specs=[
            pl.BlockSpec((_BN, _H), lambda i: (i, 0)),
            pl.BlockSpec((2, _BN, _H), lambda i: (0, i, 0)),
            pl.BlockSpec((2, _BN, 1), lambda i: (0, i, 0)),
            wspec, bspec, wspec,
            pl.BlockSpec((_H, _H), lambda i: (0, 1)),
            bspec, wspec, bspec,
        ],
        out_specs=pl.BlockSpec((_BN, _H), lambda i: (i, 0)),
        out_shape=jax.ShapeDtypeStruct((_N, _H), jnp.float32),
    )(h, s, d, w2, b2r, w3, w3, b3r, w4, b4r)


def _bcast_lane(vec, lane):
    idx = jnp.full((16, 1), lane, dtype=jnp.int32)
    return lax.gather(
        vec, idx,
        dimension_numbers=lax.GatherDimensionNumbers(
            offset_dims=(), collapsed_slice_dims=(0,), start_index_map=(0,)),
        slice_sizes=(1,),
        mode=lax.GatherScatterMode.PROMISE_IN_BOUNDS)


def _edge_body(a_hbm, b_hbm, src_hbm, dst_hbm, ea_hbm, w1c_hbm, s_out, d_out,
               src0, src1, src2, dst0, dst1, dst2, ea0, ea1, ea2,
               ab0, ab1, ab2, dz_v, qd_v, w1c_v, s_sh, d_sh,
               semi, sema, semb, semt, semd):
    cid = lax.axis_index("c")
    sid = lax.axis_index("s")
    wid = sid * 2 + cid

    srcs = [src0, src1, src2]
    dsts = [dst0, dst1, dst2]
    eas = [ea0, ea1, ea2]
    abs_ = [ab0, ab1, ab2]

    zero16 = jnp.zeros((16,), jnp.float32)
    ones16 = jnp.ones((16,), jnp.float32)
    iota16 = jnp.arange(16, dtype=jnp.int32)

    def _zrow(e, _):
        for j in range(_H // 16):
            ab0[e, pl.ds(j * 16, 16)] = zero16
            dz_v[e, pl.ds(j * 16, 16)] = zero16
        return ()
    lax.fori_loop(0, _CHUNK, _zrow, ())
    row0 = pl.multiple_of(sid * _RPT, 8)
    for k in range(7):
        pltpu.sync_copy(ab0, s_sh.at[pl.ds(row0 + k * _CHUNK, _CHUNK)])
    pltpu.sync_copy(ab0.at[pl.ds(0, _RPT - 7 * _CHUNK)],
                    s_sh.at[pl.ds(row0 + 7 * _CHUNK, _RPT - 7 * _CHUNK)])

    @pl.when(sid == 15)
    def _zero_tail():
        pltpu.sync_copy(ab0.at[pl.ds(0, _RTAIL)],
                        s_sh.at[pl.ds(16 * _RPT, _RTAIL)])

    @pl.when(sid == 0)
    def _zero_deg():
        pltpu.sync_copy(ab0, d_sh)

    pltpu.sync_copy(w1c_hbm, w1c_v)
    w1cs = [w1c_v[pl.ds(j * 16, 16)] for j in range(_H // 16)]

    plsc.subcore_barrier()

    def _base(c):
        return pl.multiple_of(wid * _EPT + c * _CHUNK, 8)

    def idx_issue(c, p):
        base = _base(c)
        pltpu.async_copy(src_hbm.at[pl.ds(base, _CHUNK)], srcs[p], semi)
        pltpu.async_copy(dst_hbm.at[pl.ds(base, _CHUNK)], dsts[p], semi)
        pltpu.async_copy(ea_hbm.at[pl.ds(base, _CHUNK)], eas[p], semi)

    def idx_wait(c, p):
        base = _base(c)
        pltpu.make_async_copy(src_hbm.at[pl.ds(base, _CHUNK)], srcs[p],
                              semi).wait()
        pltpu.make_async_copy(dst_hbm.at[pl.ds(base, _CHUNK)], dsts[p],
                              semi).wait()
        pltpu.make_async_copy(ea_hbm.at[pl.ds(base, _CHUNK)], eas[p],
                              semi).wait()

    def gather_a_issue(p):
        pltpu.async_copy(a_hbm.at[srcs[p]], abs_[p], sema)

    def gather_a_wait(p):
        pltpu.make_async_copy(a_hbm.at[srcs[p]], abs_[p], sema).wait()

    def gather_b_issue(p):
        pltpu.async_copy(b_hbm.at[dsts[p]], abs_[p], semb, add=True)

    def gather_b_wait(p):
        pltpu.make_async_copy(b_hbm.at[dsts[p]], abs_[p], semb).wait()

    def compute(p):
        def group_body(g, _):
            e0 = pl.multiple_of(g * 16, 16)
            ea_g = eas[p][pl.ds(e0, 16)]
            dst_g = dsts[p][pl.ds(e0, 16)]
            rows = iota16 + e0
            cols = jnp.bitwise_and(dst_g, _H - 1)
            plsc.store_scatter(dz_v, [rows, cols], ones16)
            qd_v[pl.ds(e0, 16)] = jnp.right_shift(dst_g, 7)
            for e16 in range(16):
                eab = _bcast_lane(ea_g, e16)
                e = e0 + e16
                for j in range(_H // 16):
                    v = abs_[p][e, pl.ds(j * 16, 16)] + eab * w1cs[j]
                    abs_[p][e, pl.ds(j * 16, 16)] = jnp.maximum(v, 0.0)
            return ()
        lax.fori_loop(0, _NG, group_body, ())

    def dz_scatter_issue():
        pltpu.async_copy(dz_v, d_sh.at[qd_v], semd, add=True)

    def dz_scatter_wait():
        pltpu.make_async_copy(dz_v, d_sh.at[qd_v], semd).wait()

    def dz_clear(p):
        for g in range(_NG):
            e0 = g * 16
            dst_g = dsts[p][pl.ds(e0, 16)]
            rows = iota16 + e0
            cols = jnp.bitwise_and(dst_g, _H - 1)
            plsc.store_scatter(dz_v, [rows, cols], zero16)

    def t_scatter_issue(p):
        pltpu.async_copy(abs_[p], s_sh.at[dsts[p]], semt, add=True)

    def t_scatter_wait(p):
        pltpu.make_async_copy(abs_[p], s_sh.at[dsts[p]], semt).wait()

    idx_issue(0, 0)
    idx_wait(0, 0)
    gather_a_issue(0)
    gather_a_wait(0)
    gather_b_issue(0)
    idx_issue(1, 1)
    idx_wait(1, 1)
    gather_a_issue(1)
    gather_b_wait(0)
    compute(0)
    dz_scatter_issue()
    t_scatter_issue(0)
    idx_issue(2, 2)
    idx_wait(2, 2)
    gather_a_issue(2)
    gather_a_wait(1)
    gather_b_issue(1)

    def triple_body(k, _):
        for off in (1, 2, 3):
            c = 3 * k + off
            r = off % 3
            rp = (off - 1) % 3
            rn = (off + 1) % 3
            t_scatter_wait(rp)
            dz_scatter_wait()
            dz_clear(rp)

            @pl.when(c + 2 < _NCH)
            def _prefetch_idx():
                idx_issue(c + 2, rp)

            gather_a_wait(rn)
            gather_b_issue(rn)
            gather_b_wait(r)
            compute(r)
            dz_scatter_issue()
            t_scatter_issue(r)

            @pl.when(c + 2 < _NCH)
            def _next_gather():
                idx_wait(c + 2, rp)
                gather_a_issue(rp)
        return ()
    lax.fori_loop(0, (_NCH - 2) // 3, triple_body, ())

    t_scatter_wait(0)
    dz_scatter_wait()
    gather_b_wait(1)
    compute(1)
    dz_scatter_issue()
    t_scatter_issue(1)
    dz_scatter_wait()
    t_scatter_wait(1)

    plsc.subcore_barrier()
    pltpu.sync_copy(s_sh.at[pl.ds(row0, _RPT)],
                    s_out.at[cid, pl.ds(row0, _RPT)])

    @pl.when(sid == 15)
    def _copy_tail():
        pltpu.sync_copy(s_sh.at[pl.ds(16 * _RPT, _RTAIL)],
                        s_out.at[cid, pl.ds(16 * _RPT, _RTAIL)])

    @pl.when(sid == 0)
    def _copy_deg():
        pltpu.sync_copy(d_sh, d_out.at[cid])


@functools.cache
def _make_edge_kernel():
    return pl.kernel(
        _edge_body,
        out_type=[
            jax.ShapeDtypeStruct((2, _N, _H), jnp.float32),
            jax.ShapeDtypeStruct((2, _DR, _H), jnp.float32),
        ],
        mesh=plsc.VectorSubcoreMesh(core_axis_name="c", subcore_axis_name="s"),
        compiler_params=pltpu.CompilerParams(needs_layout_passes=False),
        scratch_types=[
            pltpu.VMEM((_CHUNK,), jnp.int32),
            pltpu.VMEM((_CHUNK,), jnp.int32),
            pltpu.VMEM((_CHUNK,), jnp.int32),
            pltpu.VMEM((_CHUNK,), jnp.int32),
            pltpu.VMEM((_CHUNK,), jnp.int32),
            pltpu.VMEM((_CHUNK,), jnp.int32),
            pltpu.VMEM((_CHUNK,), jnp.float32),
            pltpu.VMEM((_CHUNK,), jnp.float32),
            pltpu.VMEM((_CHUNK,), jnp.float32),
            pltpu.VMEM((_CHUNK, _H), jnp.float32),
            pltpu.VMEM((_CHUNK, _H), jnp.float32),
            pltpu.VMEM((_CHUNK, _H), jnp.float32),
            pltpu.VMEM((_CHUNK, _H), jnp.float32),
            pltpu.VMEM((_CHUNK,), jnp.int32),
            pltpu.VMEM((_H,), jnp.float32),
            pltpu.VMEM_SHARED((_N, _H), jnp.float32),
            pltpu.VMEM_SHARED((_DR, _H), jnp.float32),
        ] + [pltpu.SemaphoreType.DMA] * 5,
    )


def kernel(h, edge_index, edge_attr, W1, b1, W2, b2, W3, b3, W4, b4):
    src = edge_index[0].astype(jnp.int32)
    dst = edge_index[1].astype(jnp.int32)
    ea = edge_attr.reshape(_E)
    w1c = W1[:, 2 * _H]
    a, b = _pre(h, W1, b1.reshape(1, _H))
    s, d = _make_edge_kernel()(a, b, src, dst, ea, w1c)
    dflat = d.reshape(2, _DR * _H)[:, :_N].reshape(2, _N, 1)
    return _post(h, s, dflat, W2, b2.reshape(1, _H), W3,
                 b3.reshape(1, _H), W4, b4.reshape(1, _H))


# speedup vs baseline: 1.0281x; 1.0281x over previous
import functools

import jax
import jax.numpy as jnp
from jax import lax
from jax.experimental import pallas as pl
from jax.experimental.pallas import tpu as pltpu
from jax.experimental.pallas import tpu_sc as plsc

_N = 10000
_E = 320000
_H = 128
_NW = 32
_EPT = _E // _NW
_CHUNK = 80
_NCH = _EPT // _CHUNK
_NG = _CHUNK // 16
_RPT = 624
_RTAIL = _N - 16 * _RPT
_DR = 80
_BN = 2000

_PREC = lax.Precision.HIGHEST


def _dot_t(x, w):
    return lax.dot_general(x, w, (((1,), (1,)), ((), ())), precision=_PREC,
                           preferred_element_type=jnp.float32)


def _pre_body(h_ref, w1a_ref, w1b_ref, b1_ref, a_ref, b_ref):
    hb = h_ref[...]
    a_ref[...] = _dot_t(hb, w1a_ref[...]) + b1_ref[...]
    b_ref[...] = _dot_t(hb, w1b_ref[...])


def _pre(h, w1, b1r):
    return pl.pallas_call(
        _pre_body,
        grid=(_N // _BN,),
        in_specs=[
            pl.BlockSpec((_BN, _H), lambda i: (i, 0)),
            pl.BlockSpec((_H, _H), lambda i: (0, 0)),
            pl.BlockSpec((_H, _H), lambda i: (0, 1)),
            pl.BlockSpec((1, _H), lambda i: (0, 0)),
        ],
        out_specs=[pl.BlockSpec((_BN, _H), lambda i: (i, 0))] * 2,
        out_shape=[jax.ShapeDtypeStruct((_N, _H), jnp.float32)] * 2,
    )(h, w1, w1, b1r)


def _post_body(h_ref, s_ref, d_ref, w2_ref, b2_ref, w3a_ref, w3b_ref,
               b3_ref, w4_ref, b4_ref, out_ref):
    s = s_ref[0] + s_ref[1]
    deg = d_ref[0] + d_ref[1]
    agg = _dot_t(s, w2_ref[...]) + deg * b2_ref[...]
    hb = h_ref[...]
    hupd = jnp.maximum(
        _dot_t(hb, w3a_ref[...]) + _dot_t(agg, w3b_ref[...]) + b3_ref[...],
        0.0)
    out_ref[...] = hb + _dot_t(hupd, w4_ref[...]) + b4_ref[...]


def _post(h, s, d, w2, b2r, w3, b3r, w4, b4r):
    wspec = pl.BlockSpec((_H, _H), lambda i: (0, 0))
    bspec = pl.BlockSpec((1, _H), lambda i: (0, 0))
    return pl.pallas_call(
        _post_body,
        grid=(_N // _BN,),
        in_specs=[
            pl.BlockSpec((_BN, _H), lambda i: (i, 0)),
            pl.BlockSpec((2, _BN, _H), lambda i: (0, i, 0)),
            pl.BlockSpec((2, _BN, 1), lambda i: (0, i, 0)),
            wspec, bspec, wspec,
            pl.BlockSpec((_H, _H), lambda i: (0, 1)),
            bspec, wspec, bspec,
        ],
        out_specs=pl.BlockSpec((_BN, _H), lambda i: (i, 0)),
        out_shape=jax.ShapeDtypeStruct((_N, _H), jnp.float32),
    )(h, s, d, w2, b2r, w3, w3, b3r, w4, b4r)


def _bcast_lane(vec, lane):
    idx = jnp.full((16, 1), lane, dtype=jnp.int32)
    return lax.gather(
        vec, idx,
        dimension_numbers=lax.GatherDimensionNumbers(
            offset_dims=(), collapsed_slice_dims=(0,), start_index_map=(0,)),
        slice_sizes=(1,),
        mode=lax.GatherScatterMode.PROMISE_IN_BOUNDS)


def _edge_body(a_hbm, b_hbm, src_hbm, dst_hbm, ea_hbm, w1c_hbm, s_out, d_out,
               src0, src1, src2, dst0, dst1, dst2, ea0, ea1, ea2,
               ab0, ab1, ab2, dz_v, qd_v, w1c_v, s_sh, d_sh,
               semi, sema, semb, semt, semd):
    cid = lax.axis_index("c")
    sid = lax.axis_index("s")
    wid = sid * 2 + cid

    srcs = [src0, src1, src2]
    dsts = [dst0, dst1, dst2]
    eas = [ea0, ea1, ea2]
    abs_ = [ab0, ab1, ab2]

    zero16 = jnp.zeros((16,), jnp.float32)
    ones16 = jnp.ones((16,), jnp.float32)
    iota16 = jnp.arange(16, dtype=jnp.int32)

    def _zrow(e, _):
        for j in range(_H // 16):
            ab0[e, pl.ds(j * 16, 16)] = zero16
            dz_v[e, pl.ds(j * 16, 16)] = zero16
        return ()
    lax.fori_loop(0, _CHUNK, _zrow, ())
    row0 = pl.multiple_of(sid * _RPT, 8)
    for k in range(7):
        pltpu.sync_copy(ab0, s_sh.at[pl.ds(row0 + k * _CHUNK, _CHUNK)])
    pltpu.sync_copy(ab0.at[pl.ds(0, _RPT - 7 * _CHUNK)],
                    s_sh.at[pl.ds(row0 + 7 * _CHUNK, _RPT - 7 * _CHUNK)])

    @pl.when(sid == 15)
    def _zero_tail():
        pltpu.sync_copy(ab0.at[pl.ds(0, _RTAIL)],
                        s_sh.at[pl.ds(16 * _RPT, _RTAIL)])

    @pl.when(sid == 0)
    def _zero_deg():
        pltpu.sync_copy(ab0, d_sh)

    pltpu.sync_copy(w1c_hbm, w1c_v)
    w1cs = [w1c_v[pl.ds(j * 16, 16)] for j in range(_H // 16)]

    plsc.subcore_barrier()

    def _base(c):
        return pl.multiple_of(wid * _EPT + c * _CHUNK, 8)

    def idx_issue(c, p):
        base = _base(c)
        pltpu.async_copy(src_hbm.at[pl.ds(base, _CHUNK)], srcs[p], semi)
        pltpu.async_copy(dst_hbm.at[pl.ds(base, _CHUNK)], dsts[p], semi)
        pltpu.async_copy(ea_hbm.at[pl.ds(base, _CHUNK)], eas[p], semi)

    def idx_wait(c, p):
        base = _base(c)
        pltpu.make_async_copy(src_hbm.at[pl.ds(base, _CHUNK)], srcs[p],
                              semi).wait()
        pltpu.make_async_copy(dst_hbm.at[pl.ds(base, _CHUNK)], dsts[p],
                              semi).wait()
        pltpu.make_async_copy(ea_hbm.at[pl.ds(base, _CHUNK)], eas[p],
                              semi).wait()

    def gather_a_issue(p):
        pltpu.async_copy(a_hbm.at[srcs[p]], abs_[p], sema)

    def gather_a_wait(p):
        pltpu.make_async_copy(a_hbm.at[srcs[p]], abs_[p], sema).wait()

    def gather_b_issue(p):
        pltpu.async_copy(b_hbm.at[dsts[p]], abs_[p], semb, add=True)

    def gather_b_wait(p):
        pltpu.make_async_copy(b_hbm.at[dsts[p]], abs_[p], semb).wait()

    def compute(p):
        def group_body(g, _):
            e0 = pl.multiple_of(g * 16, 16)
            ea_g = eas[p][pl.ds(e0, 16)]
            dst_g = dsts[p][pl.ds(e0, 16)]
            rows = iota16 + e0
            cols = jnp.bitwise_and(dst_g, _H - 1)
            plsc.store_scatter(dz_v, [rows, cols], ones16)
            qd_v[pl.ds(e0, 16)] = jnp.right_shift(dst_g, 7)
            for e16 in range(16):
                eab = _bcast_lane(ea_g, e16)
                e = e0 + e16
                for j in range(_H // 16):
                    v = abs_[p][e, pl.ds(j * 16, 16)] + eab * w1cs[j]
                    abs_[p][e, pl.ds(j * 16, 16)] = jnp.maximum(v, 0.0)
            return ()
        lax.fori_loop(0, _NG, group_body, ())

    def dz_scatter_issue():
        pltpu.async_copy(dz_v, d_sh.at[qd_v], semd, add=True)

    def dz_scatter_wait():
        pltpu.make_async_copy(dz_v, d_sh.at[qd_v], semd).wait()

    def dz_clear(p):
        for g in range(_NG):
            e0 = g * 16
            dst_g = dsts[p][pl.ds(e0, 16)]
            rows = iota16 + e0
            cols = jnp.bitwise_and(dst_g, _H - 1)
            plsc.store_scatter(dz_v, [rows, cols], zero16)

    def t_scatter_issue(p):
        pltpu.async_copy(abs_[p], s_sh.at[dsts[p]], semt, add=True)

    def t_scatter_wait(p):
        pltpu.make_async_copy(abs_[p], s_sh.at[dsts[p]], semt).wait()

    idx_issue(0, 0)
    idx_wait(0, 0)
    gather_a_issue(0)
    gather_a_wait(0)
    gather_b_issue(0)
    idx_issue(1, 1)
    idx_wait(1, 1)
    gather_a_issue(1)
    gather_b_wait(0)
    compute(0)
    dz_scatter_issue()
    t_scatter_issue(0)
    idx_issue(2, 2)
    idx_wait(2, 2)
    gather_a_issue(2)
    gather_a_wait(1)
    gather_b_issue(1)

    def triple_body(k, _):
        for off in (1, 2, 3):
            c = 3 * k + off
            r = off % 3
            rp = (off - 1) % 3
            rn = (off + 1) % 3
            t_scatter_wait(rp)
            dz_scatter_wait()
            dz_clear(rp)

            @pl.when(c + 2 < _NCH)
            def _prefetch_idx():
                idx_issue(c + 2, rp)

            gather_a_wait(rn)
            gather_b_issue(rn)
            gather_b_wait(r)
            compute(r)
            dz_scatter_issue()
            t_scatter_issue(r)

            @pl.when(c + 2 < _NCH)
            def _next_gather():
                idx_wait(c + 2, rp)
                gather_a_issue(rp)
        return ()
    lax.fori_loop(0, (_NCH - 2) // 3, triple_body, ())

    t_scatter_wait(0)
    dz_scatter_wait()
    gather_b_wait(1)
    compute(1)
    dz_scatter_issue()
    t_scatter_issue(1)
    dz_scatter_wait()
    t_scatter_wait(1)

    plsc.subcore_barrier()
    pltpu.sync_copy(s_sh.at[pl.ds(row0, _RPT)],
                    s_out.at[cid, pl.ds(row0, _RPT)])

    @pl.when(sid == 15)
    def _copy_tail():
        pltpu.sync_copy(s_sh.at[pl.ds(16 * _RPT, _RTAIL)],
                        s_out.at[cid, pl.ds(16 * _RPT, _RTAIL)])

    @pl.when(sid == 0)
    def _copy_deg():
        pltpu.sync_copy(d_sh, d_out.at[cid])


@functools.cache
def _make_edge_kernel():
    return pl.kernel(
        _edge_body,
        out_type=[
            jax.ShapeDtypeStruct((2, _N, _H), jnp.float32),
            jax.ShapeDtypeStruct((2, _DR, _H), jnp.float32),
        ],
        mesh=plsc.VectorSubcoreMesh(core_axis_name="c", subcore_axis_name="s"),
        compiler_params=pltpu.CompilerParams(needs_layout_passes=False),
        scratch_types=[
            pltpu.VMEM((_CHUNK,), jnp.int32),
            pltpu.VMEM((_CHUNK,), jnp.int32),
            pltpu.VMEM((_CHUNK,), jnp.int32),
            pltpu.VMEM((_CHUNK,), jnp.int32),
            pltpu.VMEM((_CHUNK,), jnp.int32),
            pltpu.VMEM((_CHUNK,), jnp.int32),
            pltpu.VMEM((_CHUNK,), jnp.float32),
            pltpu.VMEM((_CHUNK,), jnp.float32),
            pltpu.VMEM((_CHUNK,), jnp.float32),
            pltpu.VMEM((_CHUNK, _H), jnp.float32),
            pltpu.VMEM((_CHUNK, _H), jnp.float32),
            pltpu.VMEM((_CHUNK, _H), jnp.float32),
            pltpu.VMEM((_CHUNK, _H), jnp.float32),
            pltpu.VMEM((_CHUNK,), jnp.int32),
            pltpu.VMEM((_H,), jnp.float32),
            pltpu.VMEM_SHARED((_N, _H), jnp.float32),
            pltpu.VMEM_SHARED((_DR, _H), jnp.float32),
        ] + [pltpu.SemaphoreType.DMA] * 5,
    )


def kernel(h, edge_index, edge_attr, W1, b1, W2, b2, W3, b3, W4, b4):
    src = edge_index[0].astype(jnp.int32)
    dst = edge_index[1].astype(jnp.int32)
    ea = edge_attr.reshape(_E)
    w1c = W1[:, 2 * _H]
    a, b = _pre(h, W1, b1.reshape(1, _H))
    s, d = _make_edge_kernel()(a, b, src, dst, ea, w1c)
    dflat = d.reshape(2, _DR * _H)[:, :_N].reshape(2, _N, 1)
    return _post(h, s, dflat, W2, b2.reshape(1, _H), W3,
                 b3.reshape(1, _H), W4, b4.reshape(1, _H))
